# baseline (speedup 1.0000x reference)
import functools

import jax
import jax.numpy as jnp
from jax import lax
from jax.experimental import pallas as pl
from jax.experimental.pallas import tpu as pltpu
from jax.experimental.pallas import tpu_sc as plsc

D_MODEL = 1024
N_HEADS = 16
D_HEAD = D_MODEL // N_HEADS
D_FF = 4096
N_EXPERTS = 8
SEQ = 2048
EPS = 1e-5

BLK = 256
EB = 512
FFC = 2048
NFFC = D_FF // FFC
PAD = SEQ + N_EXPERTS * EB
MAX_BLOCKS = PAD // EB
QB = 512

_f32 = jnp.float32
_i32 = jnp.int32


def _fold_body(inw_ref, w_ref, b_ref, inb_ref, weff_ref, beff_ref):
    inw = inw_ref[0]
    weff_ref[0] = lax.dot_general(inw, w_ref[0], (((1,), (0,)), ((), ())),
                                  preferred_element_type=_f32)
    beff_ref[0] = lax.dot_general(b_ref[0], inw, (((1,), (1,)), ((), ())),
                                  preferred_element_type=_f32) + inb_ref[0]


def _fold(in_w3, w3, b3, inb3):
    return pl.pallas_call(
        _fold_body,
        grid=(3,),
        in_specs=[
            pl.BlockSpec((1, D_MODEL, D_MODEL), lambda i: (i, 0, 0)),
            pl.BlockSpec((1, D_MODEL, D_MODEL), lambda i: (i, 0, 0)),
            pl.BlockSpec((1, 1, D_MODEL), lambda i: (i, 0, 0)),
            pl.BlockSpec((1, 1, D_MODEL), lambda i: (i, 0, 0)),
        ],
        out_specs=[
            pl.BlockSpec((1, D_MODEL, D_MODEL), lambda i: (i, 0, 0)),
            pl.BlockSpec((1, 1, D_MODEL), lambda i: (i, 0, 0)),
        ],
        out_shape=[
            jax.ShapeDtypeStruct((3, D_MODEL, D_MODEL), _f32),
            jax.ShapeDtypeStruct((3, 1, D_MODEL), _f32),
        ],
    )(in_w3, w3, b3, inb3)


def _qkv_body(x_ref, weff_ref, beff_ref, g_ref, b_ref, q_ref, k_ref, v_ref):
    xb = x_ref[...]
    mu = jnp.mean(xb, axis=-1, keepdims=True)
    var = jnp.mean((xb - mu) * (xb - mu), axis=-1, keepdims=True)
    hn = (xb - mu) * lax.rsqrt(var + EPS) * g_ref[...] + b_ref[...]
    for i, out in enumerate((q_ref, k_ref, v_ref)):
        out[...] = lax.dot_general(hn, weff_ref[i], (((1,), (1,)), ((), ())),
                                   preferred_element_type=_f32) + beff_ref[i]


def _qkv(x, weff, beff, ln1_g, ln1_b):
    return pl.pallas_call(
        _qkv_body,
        grid=(SEQ // BLK,),
        in_specs=[
            pl.BlockSpec((BLK, D_MODEL), lambda i: (i, 0)),
            pl.BlockSpec((3, D_MODEL, D_MODEL), lambda i: (0, 0, 0)),
            pl.BlockSpec((3, 1, D_MODEL), lambda i: (0, 0, 0)),
            pl.BlockSpec((1, D_MODEL), lambda i: (0, 0)),
            pl.BlockSpec((1, D_MODEL), lambda i: (0, 0)),
        ],
        out_specs=[pl.BlockSpec((BLK, D_MODEL), lambda i: (i, 0))] * 3,
        out_shape=[jax.ShapeDtypeStruct((SEQ, D_MODEL), _f32)] * 3,
    )(x, weff, beff, ln1_g, ln1_b)


def _attn_body(q_ref, k_ref, v_ref, o_ref):
    q = q_ref[0]
    k = k_ref[0]
    s = lax.dot_general(q, k, (((1,), (1,)), ((), ())),
                        preferred_element_type=_f32) * (1.0 / (D_HEAD ** 0.5))
    m = jnp.max(s, axis=-1, keepdims=True)
    p = jnp.exp(s - m)
    z = jnp.sum(p, axis=-1, keepdims=True)
    o = lax.dot_general(p, v_ref[0], (((1,), (0,)), ((), ())),
                        preferred_element_type=_f32)
    o_ref[0] = o / z


def _attention(qh, kh, vh):
    return pl.pallas_call(
        _attn_body,
        grid=(N_HEADS, SEQ // QB),
        in_specs=[
            pl.BlockSpec((1, QB, D_HEAD), lambda h, i: (h, i, 0)),
            pl.BlockSpec((1, SEQ, D_HEAD), lambda h, i: (h, 0, 0)),
            pl.BlockSpec((1, SEQ, D_HEAD), lambda h, i: (h, 0, 0)),
        ],
        out_specs=pl.BlockSpec((1, QB, D_HEAD), lambda h, i: (h, i, 0)),
        out_shape=jax.ShapeDtypeStruct((N_HEADS, SEQ, D_HEAD), _f32),
    )(qh, kh, vh)


def _post_body(a_ref, x_ref, ow_ref, ob_ref, g_ref, b_ref, wr_ref, br_ref,
               h_ref, h2b_ref, oh_ref, gate_ref):
    hb = lax.dot_general(a_ref[...], ow_ref[...], (((1,), (1,)), ((), ())),
                         preferred_element_type=_f32) + ob_ref[...] + x_ref[...]
    h_ref[...] = hb
    mu = jnp.mean(hb, axis=-1, keepdims=True)
    var = jnp.mean((hb - mu) * (hb - mu), axis=-1, keepdims=True)
    h2 = (hb - mu) * lax.rsqrt(var + EPS) * g_ref[...] + b_ref[...]
    h2b_ref[...] = h2.astype(jnp.bfloat16)
    logits = lax.dot_general(h2, wr_ref[...], (((1,), (1,)), ((), ())),
                             preferred_element_type=_f32) + br_ref[...]
    col = lax.broadcasted_iota(_i32, logits.shape, 1)
    lm = jnp.where(col < N_EXPERTS, logits, -1e30)
    m = jnp.max(lm, axis=-1, keepdims=True)
    e = jnp.where(col < N_EXPERTS, jnp.exp(lm - m), 0.0)
    z = jnp.sum(e, axis=-1, keepdims=True)
    gate_ref[...] = 1.0 / z
    oh_ref[...] = (lm == m).astype(_f32)


def _post(attn, x, out_w, out_b, ln2_g, ln2_b, wr_pad, br_pad):
    return pl.pallas_call(
        _post_body,
        grid=(SEQ // BLK,),
        in_specs=[
            pl.BlockSpec((BLK, D_MODEL), lambda i: (i, 0)),
            pl.BlockSpec((BLK, D_MODEL), lambda i: (i, 0)),
            pl.BlockSpec((D_MODEL, D_MODEL), lambda i: (0, 0)),
            pl.BlockSpec((1, D_MODEL), lambda i: (0, 0)),
            pl.BlockSpec((1, D_MODEL), lambda i: (0, 0)),
            pl.BlockSpec((1, D_MODEL), lambda i: (0, 0)),
            pl.BlockSpec((128, D_MODEL), lambda i: (0, 0)),
            pl.BlockSpec((1, 128), lambda i: (0, 0)),
        ],
        out_specs=[
            pl.BlockSpec((BLK, D_MODEL), lambda i: (i, 0)),
            pl.BlockSpec((BLK, D_MODEL), lambda i: (i, 0)),
            pl.BlockSpec((BLK, 128), lambda i: (i, 0)),
            pl.BlockSpec((BLK, 1), lambda i: (i, 0)),
        ],
        out_shape=[
            jax.ShapeDtypeStruct((SEQ, D_MODEL), _f32),
            jax.ShapeDtypeStruct((SEQ, D_MODEL), jnp.bfloat16),
            jax.ShapeDtypeStruct((SEQ, 128), _f32),
            jax.ShapeDtypeStruct((SEQ, 1), _f32),
        ],
    )(attn, x, out_w, out_b, ln2_g, ln2_b, wr_pad, br_pad)


def _plumb_body(oh_ref, dest_ref, be_ref, nv_ref):
    oh = oh_ref[...]
    row = lax.broadcasted_iota(_i32, (SEQ, SEQ), 0)
    col = lax.broadcasted_iota(_i32, (SEQ, SEQ), 1)
    tril = (row >= col).astype(_f32)
    cum = lax.dot_general(tril, oh, (((1,), (0,)), ((), ())),
                          preferred_element_type=_f32)
    counts = cum[SEQ - 1:SEQ, :]
    nb = jnp.floor((counts + (EB - 1)) / EB)
    r8 = lax.broadcasted_iota(_i32, (128, 128), 0)
    c8 = lax.broadcasted_iota(_i32, (128, 128), 1)
    m8 = (r8 <= c8).astype(_f32)
    cnb = lax.dot_general(nb, m8, (((1,), (0,)), ((), ())),
                          preferred_element_type=_f32)
    off = (cnb - nb) * EB
    rank = jnp.sum(oh * cum, axis=-1, keepdims=True) - 1.0
    dest = jnp.sum(oh * off, axis=-1, keepdims=True) + rank
    dest_ref[...] = dest.astype(_i32)
    bi = lax.broadcasted_iota(_i32, (1, 128), 1)
    be = jnp.zeros((1, 128), _i32)
    for e in range(N_EXPERTS):
        be = be + (bi >= cnb[0:1, e:e + 1].astype(_i32)).astype(_i32)
    be_ref[...] = jnp.minimum(be, N_EXPERTS - 1)
    nv_ref[...] = cnb[0:1, N_EXPERTS - 1:N_EXPERTS].astype(_i32)


def _plumb(oh):
    return pl.pallas_call(
        _plumb_body,
        out_shape=[
            jax.ShapeDtypeStruct((SEQ, 1), _i32),
            jax.ShapeDtypeStruct((1, 128), _i32),
            jax.ShapeDtypeStruct((1, 1), _i32),
        ],
    )(oh)


def _sc_permute(src, idx, out_rows, gather):
    info = plsc.get_sparse_core_info()
    nw = info.num_cores * info.num_subcores
    n = idx.shape[0]
    rows_w = n // nw
    width = src.shape[1]
    mesh = plsc.VectorSubcoreMesh(core_axis_name="c", subcore_axis_name="s")

    @functools.partial(
        pl.kernel,
        mesh=mesh,
        out_type=jax.ShapeDtypeStruct((out_rows, width), src.dtype),
        scratch_types=[
            pltpu.VMEM((rows_w,), _i32),
            pltpu.VMEM((rows_w, width), src.dtype),
            pltpu.SemaphoreType.DMA,
        ],
    )
    def k(src_hbm, idx_hbm, out_hbm, idx_v, rows_v, sem):
        wid = lax.axis_index("s") * info.num_cores + lax.axis_index("c")
        base = wid * rows_w
        pltpu.sync_copy(idx_hbm.at[pl.ds(base, rows_w)], idx_v)
        if gather:
            pltpu.async_copy(src_hbm.at[idx_v], rows_v, sem).wait()
            pltpu.sync_copy(rows_v, out_hbm.at[pl.ds(base, rows_w)])
        else:
            pltpu.sync_copy(src_hbm.at[pl.ds(base, rows_w)], rows_v)
            pltpu.async_copy(rows_v, out_hbm.at[idx_v], sem).wait()

    return k(src, idx)


def _ffn_body(be_s, nv_s, xs_ref, w1_ref, b1_ref, w2_ref, b2_ref, o_ref):
    i = pl.program_id(0)
    c = pl.program_id(1)

    @pl.when(i < nv_s[0])
    def _():
        xb = xs_ref[...]
        t = lax.dot_general(xb, w1_ref[0].astype(jnp.bfloat16),
                            (((1,), (1,)), ((), ())),
                            preferred_element_type=_f32) + b1_ref[0]
        t = jnp.maximum(t, 0.0)
        part = lax.dot_general(t.astype(jnp.bfloat16),
                               w2_ref[0].astype(jnp.bfloat16),
                               (((1,), (1,)), ((), ())),
                               preferred_element_type=_f32)

        @pl.when(c == 0)
        def _():
            o_ref[...] = (part + b2_ref[0]).astype(jnp.bfloat16)

        @pl.when(c > 0)
        def _():
            o_ref[...] = (o_ref[...].astype(_f32) + part).astype(jnp.bfloat16)


def _ffn(be, nv, xs, w1, b1r, w2, b2r):
    grid_spec = pltpu.PrefetchScalarGridSpec(
        num_scalar_prefetch=2,
        grid=(MAX_BLOCKS, NFFC),
        in_specs=[
            pl.BlockSpec((EB, D_MODEL), lambda i, c, be_s, nv_s: (i, 0)),
            pl.BlockSpec((1, FFC, D_MODEL),
                         lambda i, c, be_s, nv_s: (be_s[i], c, 0)),
            pl.BlockSpec((1, 1, FFC), lambda i, c, be_s, nv_s: (be_s[i], 0, c)),
            pl.BlockSpec((1, D_MODEL, FFC),
                         lambda i, c, be_s, nv_s: (be_s[i], 0, c)),
            pl.BlockSpec((1, 1, D_MODEL),
                         lambda i, c, be_s, nv_s: (be_s[i], 0, 0)),
        ],
        out_specs=pl.BlockSpec((EB, D_MODEL), lambda i, c, be_s, nv_s: (i, 0)),
    )
    return pl.pallas_call(
        _ffn_body,
        grid_spec=grid_spec,
        out_shape=jax.ShapeDtypeStruct((PAD, D_MODEL), jnp.bfloat16),
    )(be, nv, xs, w1, b1r, w2, b2r)


def _combine_body(h_ref, moe_ref, gate_ref, o_ref):
    o_ref[...] = h_ref[...] + gate_ref[...] * moe_ref[...].astype(_f32)


def _combine(h, moe, gate):
    return pl.pallas_call(
        _combine_body,
        grid=(SEQ // BLK,),
        in_specs=[
            pl.BlockSpec((BLK, D_MODEL), lambda i: (i, 0)),
            pl.BlockSpec((BLK, D_MODEL), lambda i: (i, 0)),
            pl.BlockSpec((BLK, 1), lambda i: (i, 0)),
        ],
        out_specs=pl.BlockSpec((BLK, D_MODEL), lambda i: (i, 0)),
        out_shape=jax.ShapeDtypeStruct((SEQ, D_MODEL), _f32),
    )(h, moe, gate)


def kernel(x, causal_mask, Wk, bk, Wq, bq, Wv, bv, in_proj_w, in_proj_b,
           out_proj_w, out_proj_b, ln1_g, ln1_b, ln2_g, ln2_b, Wr, br,
           W1, b1, W2, b2):
    del causal_mask

    w3 = jnp.stack([Wk, Wq, Wv])
    b3 = jnp.stack([bk, bq, bv]).reshape(3, 1, D_MODEL)
    inw3 = in_proj_w.reshape(3, D_MODEL, D_MODEL)
    inb3 = in_proj_b.reshape(3, 1, D_MODEL)
    weff, beff = _fold(inw3, w3, b3, inb3)

    q, k, v = _qkv(x, weff, beff, ln1_g.reshape(1, -1), ln1_b.reshape(1, -1))
    qh = q.reshape(SEQ, N_HEADS, D_HEAD).transpose(1, 0, 2)
    kh = k.reshape(SEQ, N_HEADS, D_HEAD).transpose(1, 0, 2)
    vh = v.reshape(SEQ, N_HEADS, D_HEAD).transpose(1, 0, 2)
    oh_attn = _attention(qh, kh, vh)
    attn = oh_attn.transpose(1, 0, 2).reshape(SEQ, D_MODEL)

    wr_pad = jnp.zeros((128, D_MODEL), _f32).at[:N_EXPERTS].set(Wr)
    br_pad = jnp.zeros((1, 128), _f32).at[0, :N_EXPERTS].set(br)
    h, h2b, onehot, gate = _post(attn, x, out_proj_w,
                                out_proj_b.reshape(1, -1),
                                ln2_g.reshape(1, -1), ln2_b.reshape(1, -1),
                                wr_pad, br_pad)

    dest2d, be2d, nv2d = _plumb(onehot)
    dest = dest2d.reshape(SEQ)
    be = be2d.reshape(128)[:MAX_BLOCKS]
    nv = nv2d.reshape(1)

    h2i = lax.bitcast_convert_type(h2b.reshape(SEQ, D_MODEL // 2, 2), _i32)
    xs_i = _sc_permute(h2i, dest, PAD, gather=False)
    x_sorted = lax.bitcast_convert_type(xs_i, jnp.bfloat16).reshape(PAD, D_MODEL)
    ffn_sorted = _ffn(be, nv, x_sorted, W1, b1.reshape(N_EXPERTS, 1, D_FF),
                      W2, b2.reshape(N_EXPERTS, 1, D_MODEL))
    ffn_i = lax.bitcast_convert_type(
        ffn_sorted.reshape(PAD, D_MODEL // 2, 2), _i32)
    moe_i = _sc_permute(ffn_i, dest, SEQ, gather=True)
    moe = lax.bitcast_convert_type(moe_i, jnp.bfloat16).reshape(SEQ, D_MODEL)

    return _combine(h, moe, gate)


# speedup vs baseline: 1.7296x; 1.7296x over previous
import functools

import jax
import jax.numpy as jnp
from jax import lax
from jax.experimental import pallas as pl
from jax.experimental.pallas import tpu as pltpu
from jax.experimental.pallas import tpu_sc as plsc

D_MODEL = 1024
N_HEADS = 16
D_HEAD = D_MODEL // N_HEADS
D_FF = 4096
N_EXPERTS = 8
SEQ = 2048
EPS = 1e-5

BLK = 256
EB = 512
FFC = 2048
NFFC = D_FF // FFC
PAD = SEQ + N_EXPERTS * EB
MAX_BLOCKS = PAD // EB
QB = 512

_f32 = jnp.float32
_i32 = jnp.int32


def _fold_body(inw_ref, w_ref, b_ref, inb_ref, weff_ref, beff_ref):
    inw = inw_ref[0]
    weff_ref[0] = lax.dot_general(inw, w_ref[0], (((1,), (0,)), ((), ())),
                                  preferred_element_type=_f32)
    beff_ref[0] = lax.dot_general(b_ref[0], inw, (((1,), (1,)), ((), ())),
                                  preferred_element_type=_f32) + inb_ref[0]


def _fold(in_w3, w3, b3, inb3):
    return pl.pallas_call(
        _fold_body,
        grid=(3,),
        in_specs=[
            pl.BlockSpec((1, D_MODEL, D_MODEL), lambda i: (i, 0, 0)),
            pl.BlockSpec((1, D_MODEL, D_MODEL), lambda i: (i, 0, 0)),
            pl.BlockSpec((1, 1, D_MODEL), lambda i: (i, 0, 0)),
            pl.BlockSpec((1, 1, D_MODEL), lambda i: (i, 0, 0)),
        ],
        out_specs=[
            pl.BlockSpec((1, D_MODEL, D_MODEL), lambda i: (i, 0, 0)),
            pl.BlockSpec((1, 1, D_MODEL), lambda i: (i, 0, 0)),
        ],
        out_shape=[
            jax.ShapeDtypeStruct((3, D_MODEL, D_MODEL), _f32),
            jax.ShapeDtypeStruct((3, 1, D_MODEL), _f32),
        ],
    )(in_w3, w3, b3, inb3)


def _qkv_body(x_ref, weff_ref, beff_ref, g_ref, b_ref, q_ref, k_ref, v_ref):
    xb = x_ref[...]
    mu = jnp.mean(xb, axis=-1, keepdims=True)
    var = jnp.mean((xb - mu) * (xb - mu), axis=-1, keepdims=True)
    hn = (xb - mu) * lax.rsqrt(var + EPS) * g_ref[...] + b_ref[...]
    for i, out in enumerate((q_ref, k_ref, v_ref)):
        out[...] = lax.dot_general(hn, weff_ref[i], (((1,), (1,)), ((), ())),
                                   preferred_element_type=_f32) + beff_ref[i]


def _qkv(x, weff, beff, ln1_g, ln1_b):
    return pl.pallas_call(
        _qkv_body,
        grid=(SEQ // BLK,),
        in_specs=[
            pl.BlockSpec((BLK, D_MODEL), lambda i: (i, 0)),
            pl.BlockSpec((3, D_MODEL, D_MODEL), lambda i: (0, 0, 0)),
            pl.BlockSpec((3, 1, D_MODEL), lambda i: (0, 0, 0)),
            pl.BlockSpec((1, D_MODEL), lambda i: (0, 0)),
            pl.BlockSpec((1, D_MODEL), lambda i: (0, 0)),
        ],
        out_specs=[pl.BlockSpec((BLK, D_MODEL), lambda i: (i, 0))] * 3,
        out_shape=[jax.ShapeDtypeStruct((SEQ, D_MODEL), _f32)] * 3,
    )(x, weff, beff, ln1_g, ln1_b)


def _attn_body(q_ref, k_ref, v_ref, o_ref):
    q = q_ref[0]
    k = k_ref[0]
    s = lax.dot_general(q, k, (((1,), (1,)), ((), ())),
                        preferred_element_type=_f32) * (1.0 / (D_HEAD ** 0.5))
    m = jnp.max(s, axis=-1, keepdims=True)
    p = jnp.exp(s - m)
    z = jnp.sum(p, axis=-1, keepdims=True)
    o = lax.dot_general(p, v_ref[0], (((1,), (0,)), ((), ())),
                        preferred_element_type=_f32)
    o_ref[0] = o / z


def _attention(qh, kh, vh):
    return pl.pallas_call(
        _attn_body,
        grid=(N_HEADS, SEQ // QB),
        in_specs=[
            pl.BlockSpec((1, QB, D_HEAD), lambda h, i: (h, i, 0)),
            pl.BlockSpec((1, SEQ, D_HEAD), lambda h, i: (h, 0, 0)),
            pl.BlockSpec((1, SEQ, D_HEAD), lambda h, i: (h, 0, 0)),
        ],
        out_specs=pl.BlockSpec((1, QB, D_HEAD), lambda h, i: (h, i, 0)),
        out_shape=jax.ShapeDtypeStruct((N_HEADS, SEQ, D_HEAD), _f32),
    )(qh, kh, vh)


def _post_body(a_ref, x_ref, ow_ref, ob_ref, g_ref, b_ref, wr_ref, br_ref,
               h_ref, h2b_ref, oh_ref, gate_ref):
    hb = lax.dot_general(a_ref[...], ow_ref[...], (((1,), (1,)), ((), ())),
                         preferred_element_type=_f32) + ob_ref[...] + x_ref[...]
    h_ref[...] = hb
    mu = jnp.mean(hb, axis=-1, keepdims=True)
    var = jnp.mean((hb - mu) * (hb - mu), axis=-1, keepdims=True)
    h2 = (hb - mu) * lax.rsqrt(var + EPS) * g_ref[...] + b_ref[...]
    h2b_ref[...] = h2
    logits = lax.dot_general(h2, wr_ref[...], (((1,), (1,)), ((), ())),
                             preferred_element_type=_f32) + br_ref[...]
    col = lax.broadcasted_iota(_i32, logits.shape, 1)
    lm = jnp.where(col < N_EXPERTS, logits, -1e30)
    m = jnp.max(lm, axis=-1, keepdims=True)
    e = jnp.where(col < N_EXPERTS, jnp.exp(lm - m), 0.0)
    z = jnp.sum(e, axis=-1, keepdims=True)
    gate_ref[...] = 1.0 / z
    oh_ref[...] = (lm == m).astype(_f32)


def _post(attn, x, out_w, out_b, ln2_g, ln2_b, wr_pad, br_pad):
    return pl.pallas_call(
        _post_body,
        grid=(SEQ // BLK,),
        in_specs=[
            pl.BlockSpec((BLK, D_MODEL), lambda i: (i, 0)),
            pl.BlockSpec((BLK, D_MODEL), lambda i: (i, 0)),
            pl.BlockSpec((D_MODEL, D_MODEL), lambda i: (0, 0)),
            pl.BlockSpec((1, D_MODEL), lambda i: (0, 0)),
            pl.BlockSpec((1, D_MODEL), lambda i: (0, 0)),
            pl.BlockSpec((1, D_MODEL), lambda i: (0, 0)),
            pl.BlockSpec((128, D_MODEL), lambda i: (0, 0)),
            pl.BlockSpec((1, 128), lambda i: (0, 0)),
        ],
        out_specs=[
            pl.BlockSpec((BLK, D_MODEL), lambda i: (i, 0)),
            pl.BlockSpec((BLK, D_MODEL), lambda i: (i, 0)),
            pl.BlockSpec((BLK, 128), lambda i: (i, 0)),
            pl.BlockSpec((BLK, 1), lambda i: (i, 0)),
        ],
        out_shape=[
            jax.ShapeDtypeStruct((SEQ, D_MODEL), _f32),
            jax.ShapeDtypeStruct((SEQ, D_MODEL), _f32),
            jax.ShapeDtypeStruct((SEQ, 128), _f32),
            jax.ShapeDtypeStruct((SEQ, 1), _f32),
        ],
    )(attn, x, out_w, out_b, ln2_g, ln2_b, wr_pad, br_pad)


def _plumb_body(oh_ref, dest_ref, be_ref, nv_ref):
    oh = oh_ref[...]
    row = lax.broadcasted_iota(_i32, (SEQ, SEQ), 0)
    col = lax.broadcasted_iota(_i32, (SEQ, SEQ), 1)
    tril = (row >= col).astype(_f32)
    cum = lax.dot_general(tril, oh, (((1,), (0,)), ((), ())),
                          preferred_element_type=_f32)
    counts = cum[SEQ - 1:SEQ, :]
    nb = jnp.floor((counts + (EB - 1)) / EB)
    r8 = lax.broadcasted_iota(_i32, (128, 128), 0)
    c8 = lax.broadcasted_iota(_i32, (128, 128), 1)
    m8 = (r8 <= c8).astype(_f32)
    cnb = lax.dot_general(nb, m8, (((1,), (0,)), ((), ())),
                          preferred_element_type=_f32)
    off = (cnb - nb) * EB
    rank = jnp.sum(oh * cum, axis=-1, keepdims=True) - 1.0
    dest = jnp.sum(oh * off, axis=-1, keepdims=True) + rank
    dest_ref[...] = dest.astype(_i32)
    bi = lax.broadcasted_iota(_i32, (1, 128), 1)
    be = jnp.zeros((1, 128), _i32)
    for e in range(N_EXPERTS):
        be = be + (bi >= cnb[0:1, e:e + 1].astype(_i32)).astype(_i32)
    be_ref[...] = jnp.minimum(be, N_EXPERTS - 1)
    nv_ref[...] = cnb[0:1, N_EXPERTS - 1:N_EXPERTS].astype(_i32)


def _plumb(oh):
    return pl.pallas_call(
        _plumb_body,
        out_shape=[
            jax.ShapeDtypeStruct((SEQ, 1), _i32),
            jax.ShapeDtypeStruct((1, 128), _i32),
            jax.ShapeDtypeStruct((1, 1), _i32),
        ],
    )(oh)


def _sc_permute(src, idx, out_rows, gather):
    info = plsc.get_sparse_core_info()
    nw = info.num_cores * info.num_subcores
    n = idx.shape[0]
    rows_w = n // nw
    width = src.shape[1]
    mesh = plsc.VectorSubcoreMesh(core_axis_name="c", subcore_axis_name="s")

    @functools.partial(
        pl.kernel,
        mesh=mesh,
        out_type=jax.ShapeDtypeStruct((out_rows, width), src.dtype),
        scratch_types=[
            pltpu.VMEM((rows_w,), _i32),
            pltpu.VMEM((rows_w, width), src.dtype),
            pltpu.SemaphoreType.DMA,
        ],
    )
    def k(src_hbm, idx_hbm, out_hbm, idx_v, rows_v, sem):
        wid = lax.axis_index("s") * info.num_cores + lax.axis_index("c")
        base = wid * rows_w
        pltpu.sync_copy(idx_hbm.at[pl.ds(base, rows_w)], idx_v)
        if gather:
            pltpu.async_copy(src_hbm.at[idx_v], rows_v, sem).wait()
            pltpu.sync_copy(rows_v, out_hbm.at[pl.ds(base, rows_w)])
        else:
            pltpu.sync_copy(src_hbm.at[pl.ds(base, rows_w)], rows_v)
            pltpu.async_copy(rows_v, out_hbm.at[idx_v], sem).wait()

    return k(src, idx)


def _ffn_body(be_s, nv_s, xs_ref, w1_ref, b1_ref, w2_ref, b2_ref, o_ref):
    i = pl.program_id(0)
    c = pl.program_id(1)

    @pl.when(i < nv_s[0])
    def _():
        xb = xs_ref[...].astype(jnp.bfloat16)
        t = lax.dot_general(xb, w1_ref[0].astype(jnp.bfloat16),
                            (((1,), (1,)), ((), ())),
                            preferred_element_type=_f32) + b1_ref[0]
        t = jnp.maximum(t, 0.0)
        part = lax.dot_general(t.astype(jnp.bfloat16),
                               w2_ref[0].astype(jnp.bfloat16),
                               (((1,), (1,)), ((), ())),
                               preferred_element_type=_f32)

        @pl.when(c == 0)
        def _():
            o_ref[...] = part + b2_ref[0]

        @pl.when(c > 0)
        def _():
            o_ref[...] = o_ref[...] + part


def _ffn(be, nv, xs, w1, b1r, w2, b2r):
    grid_spec = pltpu.PrefetchScalarGridSpec(
        num_scalar_prefetch=2,
        grid=(MAX_BLOCKS, NFFC),
        in_specs=[
            pl.BlockSpec((EB, D_MODEL), lambda i, c, be_s, nv_s: (i, 0)),
            pl.BlockSpec((1, FFC, D_MODEL),
                         lambda i, c, be_s, nv_s: (be_s[i], c, 0)),
            pl.BlockSpec((1, 1, FFC), lambda i, c, be_s, nv_s: (be_s[i], 0, c)),
            pl.BlockSpec((1, D_MODEL, FFC),
                         lambda i, c, be_s, nv_s: (be_s[i], 0, c)),
            pl.BlockSpec((1, 1, D_MODEL),
                         lambda i, c, be_s, nv_s: (be_s[i], 0, 0)),
        ],
        out_specs=pl.BlockSpec((EB, D_MODEL), lambda i, c, be_s, nv_s: (i, 0)),
    )
    return pl.pallas_call(
        _ffn_body,
        grid_spec=grid_spec,
        out_shape=jax.ShapeDtypeStruct((PAD, D_MODEL), _f32),
    )(be, nv, xs, w1, b1r, w2, b2r)


def _combine_body(h_ref, moe_ref, gate_ref, o_ref):
    o_ref[...] = h_ref[...] + gate_ref[...] * moe_ref[...]


def _combine(h, moe, gate):
    return pl.pallas_call(
        _combine_body,
        grid=(SEQ // BLK,),
        in_specs=[
            pl.BlockSpec((BLK, D_MODEL), lambda i: (i, 0)),
            pl.BlockSpec((BLK, D_MODEL), lambda i: (i, 0)),
            pl.BlockSpec((BLK, 1), lambda i: (i, 0)),
        ],
        out_specs=pl.BlockSpec((BLK, D_MODEL), lambda i: (i, 0)),
        out_shape=jax.ShapeDtypeStruct((SEQ, D_MODEL), _f32),
    )(h, moe, gate)


def kernel(x, causal_mask, Wk, bk, Wq, bq, Wv, bv, in_proj_w, in_proj_b,
           out_proj_w, out_proj_b, ln1_g, ln1_b, ln2_g, ln2_b, Wr, br,
           W1, b1, W2, b2):
    del causal_mask

    w3 = jnp.stack([Wk, Wq, Wv])
    b3 = jnp.stack([bk, bq, bv]).reshape(3, 1, D_MODEL)
    inw3 = in_proj_w.reshape(3, D_MODEL, D_MODEL)
    inb3 = in_proj_b.reshape(3, 1, D_MODEL)
    weff, beff = _fold(inw3, w3, b3, inb3)

    q, k, v = _qkv(x, weff, beff, ln1_g.reshape(1, -1), ln1_b.reshape(1, -1))
    qh = q.reshape(SEQ, N_HEADS, D_HEAD).transpose(1, 0, 2)
    kh = k.reshape(SEQ, N_HEADS, D_HEAD).transpose(1, 0, 2)
    vh = v.reshape(SEQ, N_HEADS, D_HEAD).transpose(1, 0, 2)
    oh_attn = _attention(qh, kh, vh)
    attn = oh_attn.transpose(1, 0, 2).reshape(SEQ, D_MODEL)

    wr_pad = jnp.zeros((128, D_MODEL), _f32).at[:N_EXPERTS].set(Wr)
    br_pad = jnp.zeros((1, 128), _f32).at[0, :N_EXPERTS].set(br)
    h, h2b, onehot, gate = _post(attn, x, out_proj_w,
                                out_proj_b.reshape(1, -1),
                                ln2_g.reshape(1, -1), ln2_b.reshape(1, -1),
                                wr_pad, br_pad)

    dest2d, be2d, nv2d = _plumb(onehot)
    dest = dest2d.reshape(SEQ)
    be = be2d.reshape(128)[:MAX_BLOCKS]
    nv = nv2d.reshape(1)

    x_sorted = _sc_permute(h2b, dest, PAD, gather=False)
    ffn_sorted = _ffn(be, nv, x_sorted, W1, b1.reshape(N_EXPERTS, 1, D_FF),
                      W2, b2.reshape(N_EXPERTS, 1, D_MODEL))
    moe = _sc_permute(ffn_sorted, dest, SEQ, gather=True)

    return _combine(h, moe, gate)
